# baseline (speedup 1.0000x reference)
import functools

import jax
import jax.numpy as jnp
from jax import lax
from jax.experimental import pallas as pl
from jax.experimental.pallas import tpu as pltpu
from jax.experimental.pallas import tpu_sc as plsc

D_MODEL = 1024
SEQ_LEN = 4096
BATCH = 4
NUM_WORKERS = 32
ROWS_PER_WORKER = SEQ_LEN // NUM_WORKERS
CHUNK = 56
TAIL = ROWS_PER_WORKER - 2 * CHUNK


def _sc_broadcast(pos_table):
    mesh = plsc.VectorSubcoreMesh(core_axis_name="c", subcore_axis_name="s")

    @functools.partial(
        pl.kernel,
        out_type=jax.ShapeDtypeStruct((BATCH, SEQ_LEN, D_MODEL), jnp.float32),
        mesh=mesh,
        scratch_types=[
            pltpu.VMEM((2, CHUNK, D_MODEL), jnp.float32),
            pltpu.SemaphoreType.DMA((3,)),
            pltpu.SemaphoreType.DMA((3,)),
        ],
    )
    def body(pos_hbm, out_hbm, buf, load_sem, store_sem):
        wid = lax.axis_index("s") * mesh.num_cores + lax.axis_index("c")
        base = wid * ROWS_PER_WORKER

        def load(r0, n, dst, i):
            return pltpu.async_copy(
                pos_hbm.at[pl.ds(base + r0, n)], dst, load_sem.at[i]
            )

        def stores(r0, n, src, i):
            return [
                pltpu.async_copy(
                    src, out_hbm.at[b, pl.ds(base + r0, n)], store_sem.at[i]
                )
                for b in range(BATCH)
            ]

        l0 = load(0, CHUNK, buf.at[0], 0)
        l1 = load(CHUNK, CHUNK, buf.at[1], 1)
        l0.wait()
        st0 = stores(0, CHUNK, buf.at[0], 0)
        l1.wait()
        st1 = stores(CHUNK, CHUNK, buf.at[1], 1)
        for cp in st0:
            cp.wait()
        tail_buf = buf.at[0, pl.ds(0, TAIL)]
        l2 = load(2 * CHUNK, TAIL, tail_buf, 2)
        l2.wait()
        st2 = stores(2 * CHUNK, TAIL, tail_buf, 2)
        for cp in st1 + st2:
            cp.wait()

    return body(pos_table)


def kernel(x, pos_table):
    del x
    return _sc_broadcast(pos_table)
